# baseline (speedup 1.0000x reference)
import functools

import jax
import jax.numpy as jnp
from jax import lax
from jax.experimental import pallas as pl
from jax.experimental.pallas import tpu as pltpu
from jax.experimental.pallas import tpu_sc as plsc

B = 16384
M = 32
NC = 2
NS = 16
NW = NC * NS
BPW = B // NW
CH = 128
NCH = BPW // CH
IDX_ROWS = B // CH


def _sc_gather(user2d, item2d, user_emb, item_emb, user_bias, item_bias):
  mesh = plsc.VectorSubcoreMesh(core_axis_name="c", subcore_axis_name="s")
  f32 = jnp.float32
  out_type = (
      jax.ShapeDtypeStruct((B, M), f32),
      jax.ShapeDtypeStruct((B, M), f32),
      jax.ShapeDtypeStruct((IDX_ROWS, CH), f32),
      jax.ShapeDtypeStruct((IDX_ROWS, CH), f32),
  )

  @functools.partial(
      pl.kernel,
      out_type=out_type,
      mesh=mesh,
      compiler_params=pltpu.CompilerParams(use_tc_tiling_on_sc=False),
      scratch_types=[
          pltpu.VMEM((NCH, CH), jnp.int32),
          pltpu.VMEM((NCH, CH), jnp.int32),
          pltpu.VMEM((BPW, M), f32),
          pltpu.VMEM((BPW, M), f32),
          pltpu.VMEM((NCH, CH), f32),
          pltpu.VMEM((NCH, CH), f32),
          pltpu.SemaphoreType.DMA,
      ],
  )
  def k(user_hbm, item_hbm, uemb_hbm, iemb_hbm, ubias_hbm, ibias_hbm,
        u_out, v_out, ub_out, ib_out,
        uidx_v, iidx_v, u_v, v_v, ub_v, ib_v, sem):
    wid = lax.axis_index("s") * NC + lax.axis_index("c")
    rowbase = wid * NCH
    pltpu.sync_copy(user_hbm.at[pl.ds(rowbase, NCH)], uidx_v)
    pltpu.sync_copy(item_hbm.at[pl.ds(rowbase, NCH)], iidx_v)
    copies = []
    for j in range(NCH):
      dst = pl.ds(j * CH, CH)
      copies.append(pltpu.async_copy(
          uemb_hbm.at[uidx_v.at[j]], u_v.at[dst], sem))
      copies.append(pltpu.async_copy(
          iemb_hbm.at[iidx_v.at[j]], v_v.at[dst], sem))
      copies.append(pltpu.async_copy(
          ubias_hbm.at[uidx_v.at[j]], ub_v.at[j], sem))
      copies.append(pltpu.async_copy(
          ibias_hbm.at[iidx_v.at[j]], ib_v.at[j], sem))
    for c in copies:
      c.wait()
    base = wid * BPW
    pltpu.sync_copy(u_v, u_out.at[pl.ds(base, BPW)])
    pltpu.sync_copy(v_v, v_out.at[pl.ds(base, BPW)])
    pltpu.sync_copy(ub_v, ub_out.at[pl.ds(rowbase, NCH)])
    pltpu.sync_copy(ib_v, ib_out.at[pl.ds(rowbase, NCH)])

  return k(user2d, item2d, user_emb, item_emb, user_bias, item_bias)


def _tc_dot(u, v, ub, ib, offset):
  def body(u_ref, v_ref, ub_ref, ib_ref, off_ref, o_ref):
    prod = jnp.sum(u_ref[...] * v_ref[...], axis=1)
    o_ref[...] = prod + ub_ref[...] + ib_ref[...] + off_ref[...]

  return pl.pallas_call(
      body,
      out_shape=jax.ShapeDtypeStruct((B,), jnp.float32),
  )(u, v, ub, ib, offset)


@jax.jit
def kernel(user, item, user_emb, item_emb, user_bias, item_bias, offset):
  user2d = user.astype(jnp.int32).reshape(IDX_ROWS, CH)
  item2d = item.astype(jnp.int32).reshape(IDX_ROWS, CH)
  u_g, v_g, ub2, ib2 = _sc_gather(
      user2d, item2d, user_emb, item_emb, user_bias, item_bias)
  return _tc_dot(u_g, v_g, ub2.reshape(B), ib2.reshape(B), offset)
